# Initial kernel scaffold; baseline (speedup 1.0000x reference)
import jax
import jax.numpy as jnp
from jax.experimental import pallas as pl


def kernel(word, pos1, pos2, word_table, pos1_table, pos2_table):
    raise NotImplementedError("write your pallas kernel here")


# speedup vs baseline: 4.6804x; 4.6804x over previous
import functools

import jax
import jax.numpy as jnp
from jax import lax
from jax.experimental import pallas as pl
from jax.experimental.pallas import tpu as pltpu
from jax.experimental.pallas import tpu_sc as plsc

_NUM_CORES = 2
_NUM_SUBCORES = 16
_NUM_WORKERS = _NUM_CORES * _NUM_SUBCORES
_CHUNK = 128

WORD_DIM = 128
POS_SIZE = 16
OUT_DIM = WORD_DIM + 2 * POS_SIZE


@functools.partial(jax.jit, static_argnames=("n_tokens",))
def _embed(word, pos1, pos2, word_table, pos1_table, pos2_table, n_tokens):
    per_w = n_tokens // _NUM_WORKERS
    n_chunks = per_w // _CHUNK
    mesh = plsc.VectorSubcoreMesh(
        core_axis_name="c", subcore_axis_name="s", num_cores=_NUM_CORES
    )

    @functools.partial(
        pl.kernel,
        out_type=jax.ShapeDtypeStruct((n_tokens, OUT_DIM), jnp.float32),
        mesh=mesh,
        scratch_types=[
            pltpu.VMEM((per_w,), jnp.int32),
            pltpu.VMEM((per_w,), jnp.int32),
            pltpu.VMEM((per_w,), jnp.int32),
            pltpu.VMEM((_CHUNK, WORD_DIM), jnp.float32),
            pltpu.VMEM((_CHUNK, POS_SIZE), jnp.float32),
            pltpu.VMEM((_CHUNK, POS_SIZE), jnp.float32),
            pltpu.SemaphoreType.DMA,
            pltpu.SemaphoreType.DMA,
            pltpu.SemaphoreType.DMA,
        ],
        compiler_params=pltpu.CompilerParams(use_tc_tiling_on_sc=False),
    )
    def emb_kernel(
        word_hbm,
        pos1_hbm,
        pos2_hbm,
        wt_hbm,
        p1t_hbm,
        p2t_hbm,
        out_hbm,
        widx,
        p1idx,
        p2idx,
        wrows,
        p1rows,
        p2rows,
        wsem,
        p1sem,
        p2sem,
    ):
        wid = lax.axis_index("s") * _NUM_CORES + lax.axis_index("c")
        base = wid * per_w
        pltpu.sync_copy(word_hbm.at[pl.ds(base, per_w)], widx)
        pltpu.sync_copy(pos1_hbm.at[pl.ds(base, per_w)], p1idx)
        pltpu.sync_copy(pos2_hbm.at[pl.ds(base, per_w)], p2idx)

        @pl.loop(0, n_chunks)
        def _chunk(c):
            off = c * _CHUNK
            cw = pltpu.async_copy(
                wt_hbm.at[widx.at[pl.ds(off, _CHUNK)]], wrows, wsem
            )
            c1 = pltpu.async_copy(
                p1t_hbm.at[p1idx.at[pl.ds(off, _CHUNK)]], p1rows, p1sem
            )
            c2 = pltpu.async_copy(
                p2t_hbm.at[p2idx.at[pl.ds(off, _CHUNK)]], p2rows, p2sem
            )
            cw.wait()
            c1.wait()
            c2.wait()
            row0 = base + off
            pltpu.sync_copy(
                wrows, out_hbm.at[pl.ds(row0, _CHUNK), pl.ds(0, WORD_DIM)]
            )
            pltpu.sync_copy(
                p1rows,
                out_hbm.at[pl.ds(row0, _CHUNK), pl.ds(WORD_DIM, POS_SIZE)],
            )
            pltpu.sync_copy(
                p2rows,
                out_hbm.at[
                    pl.ds(row0, _CHUNK), pl.ds(WORD_DIM + POS_SIZE, POS_SIZE)
                ],
            )

    return emb_kernel(word, pos1, pos2, word_table, pos1_table, pos2_table)


def kernel(word, pos1, pos2, word_table, pos1_table, pos2_table):
    b, l = word.shape
    n_tokens = b * l
    assert n_tokens % (_NUM_WORKERS * _CHUNK) == 0
    out2d = _embed(
        word.reshape(-1).astype(jnp.int32),
        pos1.reshape(-1).astype(jnp.int32),
        pos2.reshape(-1).astype(jnp.int32),
        word_table,
        pos1_table,
        pos2_table,
        n_tokens,
    )
    return out2d.reshape(b, l, OUT_DIM)


# speedup vs baseline: 4.9161x; 1.0504x over previous
import functools

import jax
import jax.numpy as jnp
from jax import lax
from jax.experimental import pallas as pl
from jax.experimental.pallas import tpu as pltpu
from jax.experimental.pallas import tpu_sc as plsc

_NUM_CORES = 2
_NUM_SUBCORES = 16
_NUM_WORKERS = _NUM_CORES * _NUM_SUBCORES
_GCHUNK = 128
_SUPER = 256

WORD_DIM = 128
POS_SIZE = 16
OUT_DIM = WORD_DIM + 2 * POS_SIZE


@functools.partial(jax.jit, static_argnames=("n_tokens",))
def _embed(word, pos1, pos2, word_table, pos1_table, pos2_table, n_tokens):
    per_w = n_tokens // _NUM_WORKERS
    n_super = per_w // _SUPER
    assert n_super % 2 == 1, "pipeline below assumes an odd superchunk count"
    mesh = plsc.VectorSubcoreMesh(
        core_axis_name="c", subcore_axis_name="s", num_cores=_NUM_CORES
    )

    @functools.partial(
        pl.kernel,
        out_type=jax.ShapeDtypeStruct((n_tokens, OUT_DIM), jnp.float32),
        mesh=mesh,
        scratch_types=[
            pltpu.VMEM((per_w,), jnp.int32),
            pltpu.VMEM((per_w,), jnp.int32),
            pltpu.VMEM((per_w,), jnp.int32),
            pltpu.VMEM((_SUPER, WORD_DIM), jnp.float32),
            pltpu.VMEM((_SUPER, POS_SIZE), jnp.float32),
            pltpu.VMEM((_SUPER, POS_SIZE), jnp.float32),
            pltpu.VMEM((_SUPER, WORD_DIM), jnp.float32),
            pltpu.VMEM((_SUPER, POS_SIZE), jnp.float32),
            pltpu.VMEM((_SUPER, POS_SIZE), jnp.float32),
            pltpu.SemaphoreType.DMA,
            pltpu.SemaphoreType.DMA,
        ],
        compiler_params=pltpu.CompilerParams(use_tc_tiling_on_sc=False),
    )
    def emb_kernel(
        word_hbm,
        pos1_hbm,
        pos2_hbm,
        wt_hbm,
        p1t_hbm,
        p2t_hbm,
        out_hbm,
        widx,
        p1idx,
        p2idx,
        wrows_a,
        p1rows_a,
        p2rows_a,
        wrows_b,
        p1rows_b,
        p2rows_b,
        sem_a,
        sem_b,
    ):
        buf_a = (wrows_a, p1rows_a, p2rows_a)
        buf_b = (wrows_b, p1rows_b, p2rows_b)
        wid = lax.axis_index("s") * _NUM_CORES + lax.axis_index("c")
        base = wid * per_w
        pltpu.sync_copy(word_hbm.at[pl.ds(base, per_w)], widx)
        pltpu.sync_copy(pos1_hbm.at[pl.ds(base, per_w)], p1idx)
        pltpu.sync_copy(pos2_hbm.at[pl.ds(base, per_w)], p2idx)

        def transfers(s, buf):
            wrows, p1rows, p2rows = buf
            pairs = []
            for g in range(_SUPER // _GCHUNK):
                off = s * _SUPER + g * _GCHUNK
                rows = pl.ds(g * _GCHUNK, _GCHUNK)
                pairs += [
                    (
                        wt_hbm.at[widx.at[pl.ds(off, _GCHUNK)]],
                        wrows.at[rows],
                    ),
                    (
                        p1t_hbm.at[p1idx.at[pl.ds(off, _GCHUNK)]],
                        p1rows.at[rows],
                    ),
                    (
                        p2t_hbm.at[p2idx.at[pl.ds(off, _GCHUNK)]],
                        p2rows.at[rows],
                    ),
                ]
            return pairs

        def issue(s, buf, sem):
            for src, dst in transfers(s, buf):
                pltpu.async_copy(src, dst, sem)

        def drain(s, buf, sem):
            for src, dst in transfers(s, buf):
                pltpu.make_async_copy(src, dst, sem).wait()

        def write(s, buf):
            wrows, p1rows, p2rows = buf
            row0 = base + s * _SUPER
            pltpu.sync_copy(
                wrows, out_hbm.at[pl.ds(row0, _SUPER), pl.ds(0, WORD_DIM)]
            )
            pltpu.sync_copy(
                p1rows,
                out_hbm.at[pl.ds(row0, _SUPER), pl.ds(WORD_DIM, POS_SIZE)],
            )
            pltpu.sync_copy(
                p2rows,
                out_hbm.at[
                    pl.ds(row0, _SUPER), pl.ds(WORD_DIM + POS_SIZE, POS_SIZE)
                ],
            )

        issue(0, buf_a, sem_a)

        @pl.loop(0, n_super - 1, step=2)
        def _body(s):
            issue(s + 1, buf_b, sem_b)
            drain(s, buf_a, sem_a)
            write(s, buf_a)
            issue(s + 2, buf_a, sem_a)
            drain(s + 1, buf_b, sem_b)
            write(s + 1, buf_b)

        drain(n_super - 1, buf_a, sem_a)
        write(n_super - 1, buf_a)

    return emb_kernel(word, pos1, pos2, word_table, pos1_table, pos2_table)


def kernel(word, pos1, pos2, word_table, pos1_table, pos2_table):
    b, l = word.shape
    n_tokens = b * l
    assert n_tokens % (_NUM_WORKERS * _SUPER) == 0
    out2d = _embed(
        word.reshape(-1).astype(jnp.int32),
        pos1.reshape(-1).astype(jnp.int32),
        pos2.reshape(-1).astype(jnp.int32),
        word_table,
        pos1_table,
        pos2_table,
        n_tokens,
    )
    return out2d.reshape(b, l, OUT_DIM)
